# baseline (speedup 1.0000x reference)
import functools

import jax
import jax.numpy as jnp
from jax import lax
from jax.experimental import pallas as pl
from jax.experimental.pallas import tpu as pltpu
from jax.experimental.pallas import tpu_sc as plsc

N_EMB = 1024
DIM = 64
COMMITMENT_COST = 0.25

_ROWS = 1152


def _tc_body(x_ref, emb_ref, idx_ref, loss_ref):
    i = pl.program_id(0)
    xt = x_ref[0]
    emb = emb_ref[...]
    mm = jnp.dot(emb, xt, preferred_element_type=jnp.float32)
    e2 = jnp.sum(emb * emb, axis=1, keepdims=True)
    x2 = jnp.sum(xt * xt, axis=0, keepdims=True)
    dist = e2 + x2 - 2.0 * mm
    mind = jnp.min(dist, axis=0, keepdims=True)
    code = jnp.broadcast_to(
        lax.broadcasted_iota(jnp.int32, (N_EMB, 1), 0).astype(jnp.float32),
        dist.shape,
    )
    idx_ref[0, 0, :] = jnp.min(
        jnp.where(dist == mind, code, jnp.float32(N_EMB)), axis=0
    ).astype(jnp.int32)

    @pl.when(i == 0)
    def _():
        loss_ref[...] = jnp.zeros_like(loss_ref)

    loss_ref[...] += jnp.broadcast_to(jnp.sum(mind), (1, 1))


def _tc_argmin(xt3, embedding):
    b, d, t = xt3.shape
    n = b * t
    idx2d, loss_sum = pl.pallas_call(
        _tc_body,
        grid=(b,),
        in_specs=[
            pl.BlockSpec((1, d, t), lambda i: (i, 0, 0)),
            pl.BlockSpec((N_EMB, d), lambda i: (0, 0)),
        ],
        out_specs=[
            pl.BlockSpec((1, 1, t), lambda i: (i, 0, 0)),
            pl.BlockSpec((1, 1), lambda i: (0, 0)),
        ],
        out_shape=[
            jax.ShapeDtypeStruct((b, 1, t), jnp.int32),
            jax.ShapeDtypeStruct((1, 1), jnp.float32),
        ],
    )(xt3, embedding)
    return idx2d.reshape(n), loss_sum[0, 0]


_SC_CHUNK = 128


def _sc_gather_fn(n_rows):
    info = plsc.get_sparse_core_info()
    nw = info.num_cores * info.num_subcores
    b_per_w = n_rows // nw
    n_chunks = b_per_w // _SC_CHUNK
    mesh = plsc.VectorSubcoreMesh(core_axis_name="c", subcore_axis_name="s")

    @functools.partial(
        pl.kernel,
        mesh=mesh,
        out_type=jax.ShapeDtypeStruct((n_rows, DIM), jnp.float32),
        scratch_types=[
            pltpu.VMEM((n_chunks, _SC_CHUNK), jnp.int32),
            pltpu.VMEM((b_per_w, DIM), jnp.float32),
            pltpu.SemaphoreType.DMA,
        ],
        compiler_params=pltpu.CompilerParams(use_tc_tiling_on_sc=False),
    )
    def gather_k(table_hbm, idx_hbm, out_hbm, idx_v, rows_v, sem):
        wid = lax.axis_index("s") * info.num_cores + lax.axis_index("c")
        base = wid * b_per_w
        pltpu.sync_copy(idx_hbm.at[wid], idx_v)
        copies = [
            pltpu.async_copy(
                table_hbm.at[idx_v.at[j]],
                rows_v.at[pl.ds(j * _SC_CHUNK, _SC_CHUNK)],
                sem,
            )
            for j in range(n_chunks)
        ]
        for c in copies:
            c.wait()
        pltpu.sync_copy(rows_v, out_hbm.at[pl.ds(base, b_per_w)])

    return gather_k


def kernel(x, embedding):
    n = x.shape[0] * x.shape[1]
    indices, loss_sum = _tc_argmin(jnp.transpose(x, (0, 2, 1)), embedding)
    idx2d = indices.reshape(32, n // (32 * _SC_CHUNK), _SC_CHUNK)
    quantized = _sc_gather_fn(n)(embedding, idx2d)
    loss = loss_sum * (COMMITMENT_COST / (n * DIM))
    return quantized.reshape(x.shape), loss, indices


# speedup vs baseline: 1.1696x; 1.1696x over previous
import functools

import jax
import jax.numpy as jnp
from jax import lax
from jax.experimental import pallas as pl
from jax.experimental.pallas import tpu as pltpu
from jax.experimental.pallas import tpu_sc as plsc

N_EMB = 1024
DIM = 64
COMMITMENT_COST = 0.25

_ROWS = 1152


_SUB = 4


def _tc_body(x_ref, emb_ref, idx_ref, loss_ref):
    i = pl.program_id(0)
    emb = emb_ref[...]
    e2 = jnp.sum(emb * emb, axis=1, keepdims=True)

    @pl.when(i == 0)
    def _():
        loss_ref[...] = jnp.zeros_like(loss_ref)

    for j in range(_SUB):
        xt = x_ref[j]
        mm = jnp.dot(emb, xt, preferred_element_type=jnp.float32)
        x2 = jnp.sum(xt * xt, axis=0, keepdims=True)
        dist = e2 + x2 - 2.0 * mm
        mind = jnp.min(dist, axis=0, keepdims=True)
        code = jnp.broadcast_to(
            lax.broadcasted_iota(jnp.int32, (N_EMB, 1), 0).astype(jnp.float32),
            dist.shape,
        )
        idx_ref[j, 0, :] = jnp.min(
            jnp.where(dist == mind, code, jnp.float32(N_EMB)), axis=0
        ).astype(jnp.int32)
        loss_ref[...] += jnp.broadcast_to(jnp.sum(mind), (1, 1))


def _tc_argmin(xt3, embedding):
    b, d, t = xt3.shape
    n = b * t
    idx2d, loss_sum = pl.pallas_call(
        _tc_body,
        grid=(b // _SUB,),
        in_specs=[
            pl.BlockSpec((_SUB, d, t), lambda i: (i, 0, 0)),
            pl.BlockSpec((N_EMB, d), lambda i: (0, 0)),
        ],
        out_specs=[
            pl.BlockSpec((_SUB, 1, t), lambda i: (i, 0, 0)),
            pl.BlockSpec((1, 1), lambda i: (0, 0)),
        ],
        out_shape=[
            jax.ShapeDtypeStruct((b, 1, t), jnp.int32),
            jax.ShapeDtypeStruct((1, 1), jnp.float32),
        ],
    )(xt3, embedding)
    return idx2d.reshape(n), loss_sum[0, 0]


_SC_CHUNK = 128


def _sc_gather_fn(n_rows):
    info = plsc.get_sparse_core_info()
    nw = info.num_cores * info.num_subcores
    b_per_w = n_rows // nw
    n_chunks = b_per_w // _SC_CHUNK
    mesh = plsc.VectorSubcoreMesh(core_axis_name="c", subcore_axis_name="s")

    @functools.partial(
        pl.kernel,
        mesh=mesh,
        out_type=jax.ShapeDtypeStruct((n_rows, DIM), jnp.float32),
        scratch_types=[
            pltpu.VMEM((n_chunks, _SC_CHUNK), jnp.int32),
            pltpu.VMEM((b_per_w, DIM), jnp.float32),
            pltpu.SemaphoreType.DMA,
        ],
        compiler_params=pltpu.CompilerParams(use_tc_tiling_on_sc=False),
    )
    def gather_k(table_hbm, idx_hbm, out_hbm, idx_v, rows_v, sem):
        wid = lax.axis_index("s") * info.num_cores + lax.axis_index("c")
        base = wid * b_per_w
        pltpu.sync_copy(idx_hbm.at[wid], idx_v)
        copies = [
            pltpu.async_copy(
                table_hbm.at[idx_v.at[j]],
                rows_v.at[pl.ds(j * _SC_CHUNK, _SC_CHUNK)],
                sem,
            )
            for j in range(n_chunks)
        ]
        for c in copies:
            c.wait()
        pltpu.sync_copy(rows_v, out_hbm.at[pl.ds(base, b_per_w)])

    return gather_k


def kernel(x, embedding):
    n = x.shape[0] * x.shape[1]
    indices, loss_sum = _tc_argmin(jnp.transpose(x, (0, 2, 1)), embedding)
    idx2d = indices.reshape(32, n // (32 * _SC_CHUNK), _SC_CHUNK)
    quantized = _sc_gather_fn(n)(embedding, idx2d)
    loss = loss_sum * (COMMITMENT_COST / (n * DIM))
    return quantized.reshape(x.shape), loss, indices


# speedup vs baseline: 1.1707x; 1.0010x over previous
import functools

import jax
import jax.numpy as jnp
from jax import lax
from jax.experimental import pallas as pl
from jax.experimental.pallas import tpu as pltpu
from jax.experimental.pallas import tpu_sc as plsc

N_EMB = 1024
DIM = 64
COMMITMENT_COST = 0.25

_ROWS = 1152


_SUB = 4


def _tc_body(x_ref, emb_ref, idx_ref, loss_ref):
    i = pl.program_id(0)
    emb = emb_ref[...]
    e2 = jnp.sum(emb * emb, axis=1, keepdims=True)

    @pl.when(i == 0)
    def _():
        loss_ref[...] = jnp.zeros_like(loss_ref)

    for j in range(_SUB):
        xt = x_ref[j]
        mm = jnp.dot(emb, xt, preferred_element_type=jnp.float32)
        x2 = jnp.sum(xt * xt, axis=0, keepdims=True)
        dist = e2 + x2 - 2.0 * mm
        mind = jnp.min(dist, axis=0, keepdims=True)
        code = jnp.broadcast_to(
            lax.broadcasted_iota(jnp.int32, (N_EMB, 1), 0).astype(jnp.float32),
            dist.shape,
        )
        idx_ref[j, 0, :] = jnp.min(
            jnp.where(dist == mind, code, jnp.float32(N_EMB)), axis=0
        ).astype(jnp.int32)
        loss_ref[...] += jnp.broadcast_to(jnp.sum(mind), (1, 1))


def _tc_argmin(xt3, embedding):
    b, d, t = xt3.shape
    n = b * t
    idx2d, loss_sum = pl.pallas_call(
        _tc_body,
        grid=(b // _SUB,),
        in_specs=[
            pl.BlockSpec((_SUB, d, t), lambda i: (i, 0, 0)),
            pl.BlockSpec((N_EMB, d), lambda i: (0, 0)),
        ],
        out_specs=[
            pl.BlockSpec((_SUB, 1, t), lambda i: (i, 0, 0)),
            pl.BlockSpec((1, 1), lambda i: (0, 0)),
        ],
        out_shape=[
            jax.ShapeDtypeStruct((b, 1, t), jnp.int32),
            jax.ShapeDtypeStruct((1, 1), jnp.float32),
        ],
    )(xt3, embedding)
    return idx2d.reshape(n), loss_sum[0, 0]


_SC_CHUNK = 128


def _sc_gather_fn(n_rows):
    info = plsc.get_sparse_core_info()
    nw = info.num_cores * info.num_subcores
    b_per_w = n_rows // nw
    n_chunks = b_per_w // _SC_CHUNK
    mesh = plsc.VectorSubcoreMesh(core_axis_name="c", subcore_axis_name="s")

    planes_per_w = b_per_w // 576

    @functools.partial(
        pl.kernel,
        mesh=mesh,
        out_type=jax.ShapeDtypeStruct((n_rows // 576, 576, DIM), jnp.float32),
        scratch_types=[
            pltpu.VMEM((n_chunks, _SC_CHUNK), jnp.int32),
            pltpu.VMEM((b_per_w, DIM), jnp.float32),
            pltpu.SemaphoreType.DMA,
        ],
        compiler_params=pltpu.CompilerParams(use_tc_tiling_on_sc=False),
    )
    def gather_k(table_hbm, idx_hbm, out_hbm, idx_v, rows_v, sem):
        wid = lax.axis_index("s") * info.num_cores + lax.axis_index("c")
        pltpu.sync_copy(idx_hbm.at[wid], idx_v)
        copies = [
            pltpu.async_copy(
                table_hbm.at[idx_v.at[j]],
                rows_v.at[pl.ds(j * _SC_CHUNK, _SC_CHUNK)],
                sem,
            )
            for j in range(n_chunks)
        ]
        for c in copies:
            c.wait()
        for p in range(planes_per_w):
            pltpu.sync_copy(
                rows_v.at[pl.ds(p * 576, 576)],
                out_hbm.at[wid * planes_per_w + p],
            )

    return gather_k


def kernel(x, embedding):
    n = x.shape[0] * x.shape[1]
    indices, loss_sum = _tc_argmin(jnp.transpose(x, (0, 2, 1)), embedding)
    idx2d = indices.reshape(32, n // (32 * _SC_CHUNK), _SC_CHUNK)
    quantized = _sc_gather_fn(n)(embedding, idx2d)
    loss = loss_sum * (COMMITMENT_COST / (n * DIM))
    return quantized, loss, indices


# speedup vs baseline: 1.1938x; 1.0197x over previous
import functools

import jax
import jax.numpy as jnp
from jax import lax
from jax.experimental import pallas as pl
from jax.experimental.pallas import tpu as pltpu
from jax.experimental.pallas import tpu_sc as plsc

N_EMB = 1024
DIM = 64
COMMITMENT_COST = 0.25

_ROWS = 1152


_SUB = 8


def _tc_body(x_ref, emb_ref, idx_ref, loss_ref):
    i = pl.program_id(0)
    emb = emb_ref[...]
    e2 = jnp.sum(emb * emb, axis=1, keepdims=True)

    @pl.when(i == 0)
    def _():
        loss_ref[...] = jnp.zeros_like(loss_ref)

    for j in range(_SUB):
        xt = x_ref[j]
        mm = jnp.dot(emb, xt, preferred_element_type=jnp.float32)
        x2 = jnp.sum(xt * xt, axis=0, keepdims=True)
        dist = e2 + x2 - 2.0 * mm
        mind = jnp.min(dist, axis=0, keepdims=True)
        code = jnp.broadcast_to(
            lax.broadcasted_iota(jnp.int32, (N_EMB, 1), 0).astype(jnp.float32),
            dist.shape,
        )
        idx_ref[j, 0, :] = jnp.min(
            jnp.where(dist == mind, code, jnp.float32(N_EMB)), axis=0
        ).astype(jnp.int32)
        loss_ref[...] += jnp.broadcast_to(jnp.sum(mind), (1, 1))


def _tc_argmin(xt3, embedding):
    b, d, t = xt3.shape
    n = b * t
    idx2d, loss_sum = pl.pallas_call(
        _tc_body,
        grid=(b // _SUB,),
        in_specs=[
            pl.BlockSpec((_SUB, d, t), lambda i: (i, 0, 0)),
            pl.BlockSpec((N_EMB, d), lambda i: (0, 0)),
        ],
        out_specs=[
            pl.BlockSpec((_SUB, 1, t), lambda i: (i, 0, 0)),
            pl.BlockSpec((1, 1), lambda i: (0, 0)),
        ],
        out_shape=[
            jax.ShapeDtypeStruct((b, 1, t), jnp.int32),
            jax.ShapeDtypeStruct((1, 1), jnp.float32),
        ],
    )(xt3, embedding)
    return idx2d.reshape(n), loss_sum[0, 0]


_SC_CHUNK = 128


def _sc_gather_fn(n_rows):
    info = plsc.get_sparse_core_info()
    nw = info.num_cores * info.num_subcores
    b_per_w = n_rows // nw
    n_chunks = b_per_w // _SC_CHUNK
    mesh = plsc.VectorSubcoreMesh(core_axis_name="c", subcore_axis_name="s")

    planes_per_w = b_per_w // 576

    @functools.partial(
        pl.kernel,
        mesh=mesh,
        out_type=jax.ShapeDtypeStruct((n_rows // 576, 576, DIM), jnp.float32),
        scratch_types=[
            pltpu.VMEM((n_chunks, _SC_CHUNK), jnp.int32),
            pltpu.VMEM((b_per_w, DIM), jnp.float32),
            pltpu.SemaphoreType.DMA,
        ],
        compiler_params=pltpu.CompilerParams(use_tc_tiling_on_sc=False),
    )
    def gather_k(table_hbm, idx_hbm, out_hbm, idx_v, rows_v, sem):
        wid = lax.axis_index("s") * info.num_cores + lax.axis_index("c")
        pltpu.sync_copy(idx_hbm.at[wid], idx_v)
        copies = [
            pltpu.async_copy(
                table_hbm.at[idx_v.at[j]],
                rows_v.at[pl.ds(j * _SC_CHUNK, _SC_CHUNK)],
                sem,
            )
            for j in range(n_chunks)
        ]
        for c in copies:
            c.wait()
        for p in range(planes_per_w):
            pltpu.sync_copy(
                rows_v.at[pl.ds(p * 576, 576)],
                out_hbm.at[wid * planes_per_w + p],
            )

    return gather_k


def kernel(x, embedding):
    n = x.shape[0] * x.shape[1]
    indices, loss_sum = _tc_argmin(jnp.transpose(x, (0, 2, 1)), embedding)
    idx2d = indices.reshape(32, n // (32 * _SC_CHUNK), _SC_CHUNK)
    quantized = _sc_gather_fn(n)(embedding, idx2d)
    loss = loss_sum * (COMMITMENT_COST / (n * DIM))
    return quantized, loss, indices
